# baseline (speedup 1.0000x reference)
import functools

import jax
import jax.numpy as jnp
from jax import lax
from jax.experimental import pallas as pl
from jax.experimental.pallas import tpu as pltpu
from jax.experimental.pallas import tpu_sc as plsc

BATCH = 16384
EMB = 64

_NC = 2
_NS = 16
_NW = _NC * _NS
_BPW = BATCH // _NW
_CHUNK = 128
_NCH = _BPW // _CHUNK


def _sc_gather(pid, cid, P, C):
    mesh = plsc.VectorSubcoreMesh(
        core_axis_name="c", subcore_axis_name="s",
        num_cores=_NC, num_subcores=_NS)

    @functools.partial(
        pl.kernel,
        out_type=(jax.ShapeDtypeStruct((BATCH, EMB), jnp.float32),
                  jax.ShapeDtypeStruct((BATCH, EMB), jnp.float32)),
        mesh=mesh,
        scratch_types=[
            pltpu.VMEM((_NCH, _CHUNK), jnp.int32),
            pltpu.VMEM((_NCH, _CHUNK), jnp.int32),
            pltpu.VMEM((_BPW, EMB), jnp.float32),
            pltpu.VMEM((_BPW, EMB), jnp.float32),
            pltpu.SemaphoreType.DMA,
            pltpu.SemaphoreType.DMA,
        ],
        compiler_params=pltpu.CompilerParams(use_tc_tiling_on_sc=False),
    )
    def gather_kernel(pid_hbm, cid_hbm, p_hbm, c_hbm, pe_hbm, ce_hbm,
                      idx_p, idx_c, rows_p, rows_c, sem_p, sem_c):
        wid = lax.axis_index("s") * _NC + lax.axis_index("c")
        base = wid * _BPW
        row0 = wid * _NCH
        pltpu.sync_copy(pid_hbm.at[pl.ds(row0, _NCH)], idx_p)
        pltpu.sync_copy(cid_hbm.at[pl.ds(row0, _NCH)], idx_c)
        copies = []
        for j in range(_NCH):
            copies.append(pltpu.async_copy(
                p_hbm.at[idx_p.at[j]],
                rows_p.at[pl.ds(j * _CHUNK, _CHUNK)], sem_p))
            copies.append(pltpu.async_copy(
                c_hbm.at[idx_c.at[j]],
                rows_c.at[pl.ds(j * _CHUNK, _CHUNK)], sem_c))
        for cp in copies:
            cp.wait()
        pltpu.sync_copy(rows_p, pe_hbm.at[pl.ds(base, _BPW)])
        pltpu.sync_copy(rows_c, ce_hbm.at[pl.ds(base, _BPW)])

    return gather_kernel(pid, cid, P, C)


_BLK = 2048


def _mlp_body(pe_ref, ce_ref, w1a_ref, w1b_ref, b1_ref, w2_ref, b2_ref,
              w3_ref, b3_ref, w4_ref, out_ref):
    h = jnp.dot(pe_ref[...], w1a_ref[...], preferred_element_type=jnp.float32)
    h = h + jnp.dot(ce_ref[...], w1b_ref[...],
                    preferred_element_type=jnp.float32)
    h = jnp.maximum(h + b1_ref[...], 0.0)
    h = jnp.dot(h, w2_ref[...], preferred_element_type=jnp.float32)
    h = jnp.maximum(h + b2_ref[...], 0.0)
    h = jnp.dot(h, w3_ref[...], preferred_element_type=jnp.float32)
    h = jnp.maximum(h + b3_ref[...], 0.0)
    o = jnp.sum(h * w4_ref[...], axis=1)
    out_ref[...] = 5.0 / (1.0 + jnp.exp(-o))


def _tc_mlp(pe, ce, w1a, w1b, b1, w2, b2, w3, b3, w4):
    grid = (BATCH // _BLK,)
    full = lambda shape: pl.BlockSpec(shape, lambda i: (0,) * len(shape))
    return pl.pallas_call(
        _mlp_body,
        grid=grid,
        in_specs=[
            pl.BlockSpec((_BLK, EMB), lambda i: (i, 0)),
            pl.BlockSpec((_BLK, EMB), lambda i: (i, 0)),
            full((EMB, 128)), full((EMB, 128)), full((1, 128)),
            full((128, 128)), full((1, 128)),
            full((128, 128)), full((1, 128)),
            full((1, 128)),
        ],
        out_specs=pl.BlockSpec((_BLK,), lambda i: (i,)),
        out_shape=jax.ShapeDtypeStruct((BATCH,), jnp.float32),
    )(pe, ce, w1a, w1b, b1, w2, b2, w3, b3, w4)


def kernel(profile_ids, component_ids, P, C, W1, b1, W2, b2, W3, b3, W4, b4):
    pid = profile_ids.astype(jnp.int32).reshape(_NW * _NCH, _CHUNK)
    cid = component_ids.astype(jnp.int32).reshape(_NW * _NCH, _CHUNK)
    pe, ce = _sc_gather(pid, cid, P, C)

    w1a = W1[:, :EMB].T
    w1b = W1[:, EMB:].T
    b1r = b1.reshape(1, 128)
    w2t = jnp.zeros((128, 128), jnp.float32).at[:, :64].set(W2.T)
    b2r = jnp.zeros((1, 128), jnp.float32).at[0, :64].set(b2)
    w3t = jnp.zeros((128, 128), jnp.float32).at[:64, :32].set(W3.T)
    b3r = (jnp.zeros((1, 128), jnp.float32).at[0, :32].set(b3)
           .at[0, 32].set(1.0))
    w4r = (jnp.zeros((1, 128), jnp.float32).at[0, :32].set(W4[0])
           .at[0, 32].set(b4[0]))
    return _tc_mlp(pe, ce, w1a, w1b, b1r, w2t, b2r, w3t, b3r, w4r)


# speedup vs baseline: 1.0052x; 1.0052x over previous
import functools

import jax
import jax.numpy as jnp
from jax import lax
from jax.experimental import pallas as pl
from jax.experimental.pallas import tpu as pltpu
from jax.experimental.pallas import tpu_sc as plsc

BATCH = 16384
EMB = 64

_NC = 2
_NS = 16
_NW = _NC * _NS
_BPW = BATCH // _NW
_CHUNK = 128
_NCH = _BPW // _CHUNK


def _sc_gather(pid2, cid2, P2, C2):
    mesh = plsc.VectorSubcoreMesh(
        core_axis_name="c", subcore_axis_name="s",
        num_cores=_NC, num_subcores=_NS)

    @functools.partial(
        pl.kernel,
        out_type=(jax.ShapeDtypeStruct((BATCH, 2 * EMB), jnp.float32),
                  jax.ShapeDtypeStruct((BATCH, 2 * EMB), jnp.float32)),
        mesh=mesh,
        scratch_types=[
            pltpu.VMEM((8, _CHUNK), jnp.int32),
            pltpu.VMEM((_BPW, 2 * EMB), jnp.float32),
            pltpu.SemaphoreType.DMA,
        ],
    )
    def gather_kernel(pid_hbm, cid_hbm, p_hbm, c_hbm, pf_hbm, cf_hbm,
                      idx, rows, sem):
        wid = lax.axis_index("s") * _NC + lax.axis_index("c")
        base = wid * _BPW
        row0 = wid * _NCH
        pltpu.sync_copy(pid_hbm.at[pl.ds(row0, _NCH)], idx.at[pl.ds(0, _NCH)])
        pltpu.sync_copy(cid_hbm.at[pl.ds(row0, _NCH)],
                        idx.at[pl.ds(_NCH, _NCH)])
        copies = [pltpu.async_copy(
            p_hbm.at[idx.at[j]],
            rows.at[pl.ds(j * _CHUNK, _CHUNK)], sem) for j in range(_NCH)]
        for cp in copies:
            cp.wait()
        pltpu.sync_copy(rows, pf_hbm.at[pl.ds(base, _BPW)])
        copies = [pltpu.async_copy(
            c_hbm.at[idx.at[_NCH + j]],
            rows.at[pl.ds(j * _CHUNK, _CHUNK)], sem) for j in range(_NCH)]
        for cp in copies:
            cp.wait()
        pltpu.sync_copy(rows, cf_hbm.at[pl.ds(base, _BPW)])

    return gather_kernel(pid2, cid2, P2, C2)


_BLK = 2048


def _mlp_body(pf_ref, cf_ref, psel_ref, csel_ref, w1a_ref, w1b_ref, b1_ref,
              w2_ref, b2_ref, w3_ref, b3_ref, w4_ref, out_ref):
    pe = jnp.where(psel_ref[...] > 0, pf_ref[:, EMB:], pf_ref[:, :EMB])
    ce = jnp.where(csel_ref[...] > 0, cf_ref[:, EMB:], cf_ref[:, :EMB])
    h = jnp.dot(pe, w1a_ref[...], preferred_element_type=jnp.float32)
    h = h + jnp.dot(ce, w1b_ref[...], preferred_element_type=jnp.float32)
    h = jnp.maximum(h + b1_ref[...], 0.0)
    h = jnp.dot(h, w2_ref[...], preferred_element_type=jnp.float32)
    h = jnp.maximum(h + b2_ref[...], 0.0)
    h = jnp.dot(h, w3_ref[...], preferred_element_type=jnp.float32)
    h = jnp.maximum(h + b3_ref[...], 0.0)
    o = jnp.sum(h * w4_ref[...], axis=1)
    out_ref[...] = 5.0 / (1.0 + jnp.exp(-o))


def _tc_mlp(pf, cf, psel, csel, w1a, w1b, b1, w2, b2, w3, b3, w4):
    grid = (BATCH // _BLK,)
    full = lambda shape: pl.BlockSpec(shape, lambda i: (0,) * len(shape))
    return pl.pallas_call(
        _mlp_body,
        grid=grid,
        in_specs=[
            pl.BlockSpec((_BLK, 2 * EMB), lambda i: (i, 0)),
            pl.BlockSpec((_BLK, 2 * EMB), lambda i: (i, 0)),
            pl.BlockSpec((_BLK, 1), lambda i: (i, 0)),
            pl.BlockSpec((_BLK, 1), lambda i: (i, 0)),
            full((EMB, 128)), full((EMB, 128)), full((1, 128)),
            full((128, 128)), full((1, 128)),
            full((128, 128)), full((1, 128)),
            full((1, 128)),
        ],
        out_specs=pl.BlockSpec((_BLK,), lambda i: (i,)),
        out_shape=jax.ShapeDtypeStruct((BATCH,), jnp.float32),
    )(pf, cf, psel, csel, w1a, w1b, b1, w2, b2, w3, b3, w4)


def kernel(profile_ids, component_ids, P, C, W1, b1, W2, b2, W3, b3, W4, b4):
    pid = profile_ids.astype(jnp.int32)
    cid = component_ids.astype(jnp.int32)
    pid2 = (pid // 2).reshape(_NW * _NCH, _CHUNK)
    cid2 = (cid // 2).reshape(_NW * _NCH, _CHUNK)
    psel = (pid % 2).astype(jnp.int32).reshape(BATCH, 1)
    csel = (cid % 2).astype(jnp.int32).reshape(BATCH, 1)
    P2 = P.reshape(-1, 2 * EMB)
    C2 = C.reshape(-1, 2 * EMB)
    pf, cf = _sc_gather(pid2, cid2, P2, C2)

    w1a = W1[:, :EMB].T
    w1b = W1[:, EMB:].T
    b1r = b1.reshape(1, 128)
    w2t = jnp.zeros((128, 128), jnp.float32).at[:, :64].set(W2.T)
    b2r = jnp.zeros((1, 128), jnp.float32).at[0, :64].set(b2)
    w3t = jnp.zeros((128, 128), jnp.float32).at[:64, :32].set(W3.T)
    b3r = (jnp.zeros((1, 128), jnp.float32).at[0, :32].set(b3)
           .at[0, 32].set(1.0))
    w4r = (jnp.zeros((1, 128), jnp.float32).at[0, :32].set(W4[0])
           .at[0, 32].set(b4[0]))
    return _tc_mlp(pf, cf, psel, csel, w1a, w1b, b1r, w2t, b2r, w3t, b3r, w4r)


# speedup vs baseline: 1.9772x; 1.9671x over previous
import functools

import jax
import jax.numpy as jnp
from jax import lax
from jax.experimental import pallas as pl
from jax.experimental.pallas import tpu as pltpu
from jax.experimental.pallas import tpu_sc as plsc

BATCH = 16384
EMB = 64

_NC = 2
_NS = 16
_NW = _NC * _NS
_BPW = BATCH // _NW
_CHUNK = 128
_NCH = _BPW // _CHUNK


def _sc_gather(pid2, cid2, P2, C2):
    mesh = plsc.VectorSubcoreMesh(
        core_axis_name="c", subcore_axis_name="s",
        num_cores=_NC, num_subcores=_NS)

    @functools.partial(
        pl.kernel,
        out_type=(jax.ShapeDtypeStruct((BATCH, 2 * EMB), jnp.float32),
                  jax.ShapeDtypeStruct((BATCH, 2 * EMB), jnp.float32)),
        mesh=mesh,
        scratch_types=[
            pltpu.VMEM((8, _CHUNK), jnp.int32),
            pltpu.VMEM((_BPW, 2 * EMB), jnp.float32),
            pltpu.SemaphoreType.DMA,
        ],
    )
    def gather_kernel(pid_hbm, cid_hbm, p_hbm, c_hbm, pf_hbm, cf_hbm,
                      idx, rows, sem):
        wid = lax.axis_index("s") * _NC + lax.axis_index("c")
        base = wid * _BPW
        row0 = wid * _NCH
        pltpu.sync_copy(pid_hbm.at[pl.ds(row0, _NCH)], idx.at[pl.ds(0, _NCH)])
        pltpu.sync_copy(cid_hbm.at[pl.ds(row0, _NCH)],
                        idx.at[pl.ds(_NCH, _NCH)])
        copies = [pltpu.async_copy(
            p_hbm.at[idx.at[j]],
            rows.at[pl.ds(j * _CHUNK, _CHUNK)], sem) for j in range(_NCH)]
        for cp in copies:
            cp.wait()
        pltpu.sync_copy(rows, pf_hbm.at[pl.ds(base, _BPW)])
        copies = [pltpu.async_copy(
            c_hbm.at[idx.at[_NCH + j]],
            rows.at[pl.ds(j * _CHUNK, _CHUNK)], sem) for j in range(_NCH)]
        for cp in copies:
            cp.wait()
        pltpu.sync_copy(rows, cf_hbm.at[pl.ds(base, _BPW)])

    return gather_kernel(pid2, cid2, P2, C2)


_CONV_BLK = 4096


def _conv_body(a_ref, b_ref, out_ref):
    out_ref[...] = jnp.concatenate([a_ref[...].T, b_ref[...].T], axis=1)


def _dual_r(n):
    return ((n + _CONV_BLK) // 2 + _CONV_BLK - 1) // _CONV_BLK * _CONV_BLK


def _tc_pairrows(pt, r):
    nblk = r // _CONV_BLK
    return pl.pallas_call(
        _conv_body,
        grid=(nblk,),
        in_specs=[
            pl.BlockSpec((EMB, _CONV_BLK), lambda i: (0, i)),
            pl.BlockSpec((EMB, _CONV_BLK), lambda i: (0, i + nblk - 1)),
        ],
        out_specs=pl.BlockSpec((_CONV_BLK, 2 * EMB), lambda i: (i, 0)),
        out_shape=jax.ShapeDtypeStruct((r, 2 * EMB), jnp.float32),
    )(pt, pt)


_BLK = 2048


def _mlp_body(pf_ref, cf_ref, psel_ref, csel_ref, w1a_ref, w1b_ref, b1_ref,
              w2_ref, b2_ref, w3_ref, b3_ref, w4_ref, out_ref):
    pe = jnp.where(psel_ref[...] > 0, pf_ref[:, EMB:], pf_ref[:, :EMB])
    ce = jnp.where(csel_ref[...] > 0, cf_ref[:, EMB:], cf_ref[:, :EMB])
    h = jnp.dot(pe, w1a_ref[...], preferred_element_type=jnp.float32)
    h = h + jnp.dot(ce, w1b_ref[...], preferred_element_type=jnp.float32)
    h = jnp.maximum(h + b1_ref[...], 0.0)
    h = jnp.dot(h, w2_ref[...], preferred_element_type=jnp.float32)
    h = jnp.maximum(h + b2_ref[...], 0.0)
    h = jnp.dot(h, w3_ref[...], preferred_element_type=jnp.float32)
    h = jnp.maximum(h + b3_ref[...], 0.0)
    o = jnp.sum(h * w4_ref[...], axis=1)
    out_ref[...] = 5.0 / (1.0 + jnp.exp(-o))


def _tc_mlp(pf, cf, psel, csel, w1a, w1b, b1, w2, b2, w3, b3, w4):
    grid = (BATCH // _BLK,)
    full = lambda shape: pl.BlockSpec(shape, lambda i: (0,) * len(shape))
    return pl.pallas_call(
        _mlp_body,
        grid=grid,
        in_specs=[
            pl.BlockSpec((_BLK, 2 * EMB), lambda i: (i, 0)),
            pl.BlockSpec((_BLK, 2 * EMB), lambda i: (i, 0)),
            pl.BlockSpec((_BLK, 1), lambda i: (i, 0)),
            pl.BlockSpec((_BLK, 1), lambda i: (i, 0)),
            full((EMB, 128)), full((EMB, 128)), full((1, 128)),
            full((128, 128)), full((1, 128)),
            full((128, 128)), full((1, 128)),
            full((1, 128)),
        ],
        out_specs=pl.BlockSpec((_BLK,), lambda i: (i,)),
        out_shape=jax.ShapeDtypeStruct((BATCH,), jnp.float32),
    )(pf, cf, psel, csel, w1a, w1b, b1, w2, b2, w3, b3, w4)


def kernel(profile_ids, component_ids, P, C, W1, b1, W2, b2, W3, b3, W4, b4):
    rp = _dual_r(P.shape[0])
    rc = _dual_r(C.shape[0])
    op = rp - _CONV_BLK
    oc = rc - _CONV_BLK
    pid = profile_ids.astype(jnp.int32)
    cid = component_ids.astype(jnp.int32)
    pid2 = jnp.where(pid < rp, pid, pid - op).reshape(_NW * _NCH, _CHUNK)
    cid2 = jnp.where(cid < rc, cid, cid - oc).reshape(_NW * _NCH, _CHUNK)
    psel = (pid >= rp).astype(jnp.int32).reshape(BATCH, 1)
    csel = (cid >= rc).astype(jnp.int32).reshape(BATCH, 1)
    P2 = _tc_pairrows(P.T, rp)
    C2 = _tc_pairrows(C.T, rc)
    pf, cf = _sc_gather(pid2, cid2, P2, C2)

    w1a = W1[:, :EMB].T
    w1b = W1[:, EMB:].T
    b1r = b1.reshape(1, 128)
    w2t = jnp.zeros((128, 128), jnp.float32).at[:, :64].set(W2.T)
    b2r = jnp.zeros((1, 128), jnp.float32).at[0, :64].set(b2)
    w3t = jnp.zeros((128, 128), jnp.float32).at[:64, :32].set(W3.T)
    b3r = (jnp.zeros((1, 128), jnp.float32).at[0, :32].set(b3)
           .at[0, 32].set(1.0))
    w4r = (jnp.zeros((1, 128), jnp.float32).at[0, :32].set(W4[0])
           .at[0, 32].set(b4[0]))
    return _tc_mlp(pf, cf, psel, csel, w1a, w1b, b1r, w2t, b2r, w3t, b3r, w4r)


# speedup vs baseline: 2.1844x; 1.1048x over previous
import functools

import jax
import jax.numpy as jnp
from jax import lax
from jax.experimental import pallas as pl
from jax.experimental.pallas import tpu as pltpu
from jax.experimental.pallas import tpu_sc as plsc

BATCH = 16384
EMB = 64

_NC = 2
_NS = 16
_NW = _NC * _NS
_BPW = BATCH // _NW
_CHUNK = 128
_NCH = _BPW // _CHUNK


def _sc_gather(pid2, cid2, P2, C2):
    mesh = plsc.VectorSubcoreMesh(
        core_axis_name="c", subcore_axis_name="s",
        num_cores=_NC, num_subcores=_NS)

    @functools.partial(
        pl.kernel,
        out_type=(jax.ShapeDtypeStruct((BATCH, 2 * EMB), jnp.float32),
                  jax.ShapeDtypeStruct((BATCH, 2 * EMB), jnp.float32)),
        mesh=mesh,
        scratch_types=[
            pltpu.VMEM((8, _CHUNK), jnp.int32),
            pltpu.VMEM((_BPW, 2 * EMB), jnp.float32),
            pltpu.SemaphoreType.DMA,
        ],
    )
    def gather_kernel(pid_hbm, cid_hbm, p_hbm, c_hbm, pf_hbm, cf_hbm,
                      idx, rows, sem):
        wid = lax.axis_index("s") * _NC + lax.axis_index("c")
        base = wid * _BPW
        row0 = wid * _NCH
        pltpu.sync_copy(pid_hbm.at[pl.ds(row0, _NCH)], idx.at[pl.ds(0, _NCH)])
        pltpu.sync_copy(cid_hbm.at[pl.ds(row0, _NCH)],
                        idx.at[pl.ds(_NCH, _NCH)])
        copies = [pltpu.async_copy(
            p_hbm.at[idx.at[j]],
            rows.at[pl.ds(j * _CHUNK, _CHUNK)], sem) for j in range(_NCH)]
        for cp in copies:
            cp.wait()
        pltpu.sync_copy(rows, pf_hbm.at[pl.ds(base, _BPW)])
        copies = [pltpu.async_copy(
            c_hbm.at[idx.at[_NCH + j]],
            rows.at[pl.ds(j * _CHUNK, _CHUNK)], sem) for j in range(_NCH)]
        for cp in copies:
            cp.wait()
        pltpu.sync_copy(rows, cf_hbm.at[pl.ds(base, _BPW)])

    return gather_kernel(pid2, cid2, P2, C2)


_CONV_BLK = 8192


def _conv_body(a_ref, b_ref, out_ref):
    out_ref[...] = jnp.concatenate([a_ref[...].T, b_ref[...].T], axis=1)


def _dual_r(n):
    return ((n + _CONV_BLK) // 2 + _CONV_BLK - 1) // _CONV_BLK * _CONV_BLK


def _tc_pairrows(pt, r):
    nblk = r // _CONV_BLK
    return pl.pallas_call(
        _conv_body,
        grid=(nblk,),
        in_specs=[
            pl.BlockSpec((EMB, _CONV_BLK), lambda i: (0, i)),
            pl.BlockSpec((EMB, _CONV_BLK), lambda i: (0, i + nblk - 1)),
        ],
        out_specs=pl.BlockSpec((_CONV_BLK, 2 * EMB), lambda i: (i, 0)),
        out_shape=jax.ShapeDtypeStruct((r, 2 * EMB), jnp.float32),
    )(pt, pt)


_BLK = 2048


def _mlp_body(pf_ref, cf_ref, psel_ref, csel_ref, w1a_ref, w1b_ref, b1_ref,
              w2_ref, b2_ref, w3_ref, b3_ref, w4_ref, out_ref):
    pe = jnp.where(psel_ref[...] > 0, pf_ref[:, EMB:], pf_ref[:, :EMB])
    ce = jnp.where(csel_ref[...] > 0, cf_ref[:, EMB:], cf_ref[:, :EMB])
    h = jnp.dot(pe, w1a_ref[...], preferred_element_type=jnp.float32)
    h = h + jnp.dot(ce, w1b_ref[...], preferred_element_type=jnp.float32)
    h = jnp.maximum(h + b1_ref[...], 0.0)
    h = jnp.dot(h, w2_ref[...], preferred_element_type=jnp.float32)
    h = jnp.maximum(h + b2_ref[...], 0.0)
    h = jnp.dot(h, w3_ref[...], preferred_element_type=jnp.float32)
    h = jnp.maximum(h + b3_ref[...], 0.0)
    o = jnp.sum(h * w4_ref[...], axis=1)
    out_ref[...] = 5.0 / (1.0 + jnp.exp(-o))


def _tc_mlp(pf, cf, psel, csel, w1a, w1b, b1, w2, b2, w3, b3, w4):
    grid = (BATCH // _BLK,)
    full = lambda shape: pl.BlockSpec(shape, lambda i: (0,) * len(shape))
    return pl.pallas_call(
        _mlp_body,
        grid=grid,
        in_specs=[
            pl.BlockSpec((_BLK, 2 * EMB), lambda i: (i, 0)),
            pl.BlockSpec((_BLK, 2 * EMB), lambda i: (i, 0)),
            pl.BlockSpec((_BLK, 1), lambda i: (i, 0)),
            pl.BlockSpec((_BLK, 1), lambda i: (i, 0)),
            full((EMB, 128)), full((EMB, 128)), full((1, 128)),
            full((128, 128)), full((1, 128)),
            full((128, 128)), full((1, 128)),
            full((1, 128)),
        ],
        out_specs=pl.BlockSpec((_BLK,), lambda i: (i,)),
        out_shape=jax.ShapeDtypeStruct((BATCH,), jnp.float32),
    )(pf, cf, psel, csel, w1a, w1b, b1, w2, b2, w3, b3, w4)


def kernel(profile_ids, component_ids, P, C, W1, b1, W2, b2, W3, b3, W4, b4):
    rp = _dual_r(P.shape[0])
    rc = _dual_r(C.shape[0])
    op = rp - _CONV_BLK
    oc = rc - _CONV_BLK
    pid = profile_ids.astype(jnp.int32)
    cid = component_ids.astype(jnp.int32)
    pid2 = jnp.where(pid < rp, pid, pid - op).reshape(_NW * _NCH, _CHUNK)
    cid2 = jnp.where(cid < rc, cid, cid - oc).reshape(_NW * _NCH, _CHUNK)
    psel = (pid >= rp).astype(jnp.int32).reshape(BATCH, 1)
    csel = (cid >= rc).astype(jnp.int32).reshape(BATCH, 1)
    P2 = _tc_pairrows(P.T, rp)
    C2 = _tc_pairrows(C.T, rc)
    pf, cf = _sc_gather(pid2, cid2, P2, C2)

    w1a = W1[:, :EMB].T
    w1b = W1[:, EMB:].T
    b1r = b1.reshape(1, 128)
    w2t = jnp.zeros((128, 128), jnp.float32).at[:, :64].set(W2.T)
    b2r = jnp.zeros((1, 128), jnp.float32).at[0, :64].set(b2)
    w3t = jnp.zeros((128, 128), jnp.float32).at[:64, :32].set(W3.T)
    b3r = (jnp.zeros((1, 128), jnp.float32).at[0, :32].set(b3)
           .at[0, 32].set(1.0))
    w4r = (jnp.zeros((1, 128), jnp.float32).at[0, :32].set(W4[0])
           .at[0, 32].set(b4[0]))
    return _tc_mlp(pf, cf, psel, csel, w1a, w1b, b1r, w2t, b2r, w3t, b3r, w4r)
